# Initial kernel scaffold; baseline (speedup 1.0000x reference)
import functools

import jax
import jax.numpy as jnp
from jax import lax
from jax.experimental import pallas as pl
from jax.experimental.pallas import tpu as pltpu
from jax.experimental.pallas import tpu_sc as plsc

N = 10000
E = 320000
NFEAT = 128
HID = 16
NCLASS = 40

NC = 2
NS = 16
NW = NC * NS
B = 128
PER_TILE = -(-E // (NW * B)) * B
E_PAD = PER_TILE * NW
NB = PER_TILE // B
NBUF = 16
N_PAD = 10240
RPT = N_PAD // NS
DUMMY = N

_mesh = plsc.VectorSubcoreMesh(
    core_axis_name="c", subcore_axis_name="s", num_cores=NC)


@functools.partial(
    pl.kernel,
    out_type=jax.ShapeDtypeStruct((NC * N_PAD,), jnp.float32),
    mesh=_mesh,
    scratch_types=[
        pltpu.VMEM((NB, B), jnp.int32),
        pltpu.VMEM((B,), jnp.float32),
        pltpu.VMEM((RPT,), jnp.float32),
        pltpu.VMEM_SHARED((N_PAD,), jnp.float32),
        pltpu.SemaphoreType.DMA,
    ],
)
def _sc_degree(col_hbm, out_hbm, cidx, ones_v, bounce, acc, ssem):
    c = lax.axis_index("c")
    s = lax.axis_index("s")
    wid = c * NS + s

    one = jnp.ones((16,), jnp.float32)
    zero = jnp.zeros((16,), jnp.float32)
    for i in range(B // 16):
        ones_v[pl.ds(i * 16, 16)] = one

    def zb(i, carry):
        bounce[pl.ds(i * 16, 16)] = zero
        return carry
    lax.fori_loop(0, RPT // 16, zb, 0)
    pltpu.sync_copy(bounce, acc.at[pl.ds(s * RPT, RPT)])
    pltpu.sync_copy(col_hbm.at[pl.ds(wid * NB, NB)], cidx)
    plsc.subcore_barrier()

    def superblock(g, carry):
        j0 = g * NBUF
        hs = [pltpu.async_copy(ones_v, acc.at[cidx.at[j0 + b]], ssem, add=True)
              for b in range(NBUF)]
        for h in hs:
            h.wait()
        return carry
    lax.fori_loop(0, NB // NBUF, superblock, 0)

    plsc.subcore_barrier()
    pltpu.sync_copy(acc.at[pl.ds(s * RPT, RPT)], bounce)
    pltpu.sync_copy(bounce, out_hbm.at[pl.ds(c * N_PAD + s * RPT, RPT)])


@functools.partial(
    pl.kernel,
    out_type=jax.ShapeDtypeStruct((NC * N_PAD, HID), jnp.float32),
    mesh=_mesh,
    scratch_types=[
        pltpu.VMEM((NB, B), jnp.int32),
        pltpu.VMEM((NB, B), jnp.int32),
        pltpu.VMEM((NBUF * B, HID), jnp.float32),
        pltpu.VMEM((RPT, HID), jnp.float32),
        pltpu.VMEM_SHARED((N_PAD, HID), jnp.float32),
        pltpu.SemaphoreType.DMA,
        pltpu.SemaphoreType.DMA,
    ],
)
def _sc_scatter16(g_hbm, row_hbm, col_hbm, out_hbm,
                  ridx, cidx, rows, bounce, acc, gsem, ssem):
    c = lax.axis_index("c")
    s = lax.axis_index("s")
    wid = c * NS + s

    zero = jnp.zeros((16,), jnp.float32)

    def zb(i, carry):
        bounce[i] = zero
        return carry
    lax.fori_loop(0, RPT, zb, 0)
    pltpu.sync_copy(bounce, acc.at[pl.ds(s * RPT, RPT)])
    pltpu.sync_copy(row_hbm.at[pl.ds(wid * NB, NB)], ridx)
    pltpu.sync_copy(col_hbm.at[pl.ds(wid * NB, NB)], cidx)
    plsc.subcore_barrier()

    def superblock(g, carry):
        j0 = g * NBUF
        gathers = [pltpu.async_copy(g_hbm.at[ridx.at[j0 + b]],
                                    rows.at[pl.ds(b * B, B)], gsem)
                   for b in range(NBUF)]
        for h in gathers:
            h.wait()
        scatters = [pltpu.async_copy(rows.at[pl.ds(b * B, B)],
                                     acc.at[cidx.at[j0 + b]], ssem, add=True)
                    for b in range(NBUF)]
        for h in scatters:
            h.wait()
        return carry
    lax.fori_loop(0, NB // NBUF, superblock, 0)

    plsc.subcore_barrier()
    pltpu.sync_copy(acc.at[pl.ds(s * RPT, RPT)], bounce)
    pltpu.sync_copy(bounce, out_hbm.at[pl.ds(c * N_PAD + s * RPT, RPT)])


def _tc_head(deg0_ref, deg1_ref, x_ref, w1_ref, dis_ref, g1_ref):
    deg = deg0_ref[...] + deg1_ref[...] + 1.0
    dis = lax.rsqrt(deg)
    h1 = jnp.dot(x_ref[...], w1_ref[...], preferred_element_type=jnp.float32)
    dis_ref[...] = dis
    g1_ref[...] = h1 * dis


def _tc_mid(p0_ref, p1_ref, g1_ref, dis_ref, b1_ref, g2_ref):
    dis = dis_ref[...]
    s1 = dis * (p0_ref[...] + p1_ref[...] + g1_ref[...])
    a1 = jnp.maximum(s1 + b1_ref[...], 0.0)
    g2_ref[...] = a1 * dis


def _tc_tail(p0_ref, p1_ref, g2_ref, dis_ref, w2_ref, b2_ref, out_ref):
    s2 = dis_ref[...] * (p0_ref[...] + p1_ref[...] + g2_ref[...])
    logits = jnp.dot(s2, w2_ref[...],
                     preferred_element_type=jnp.float32) + b2_ref[...]
    m = jnp.max(logits, axis=1, keepdims=True)
    lse = jnp.log(jnp.sum(jnp.exp(logits - m), axis=1, keepdims=True)) + m
    out_ref[...] = logits - lse


def kernel(x, edge_index, W1, b1, W2, b2):
    row = edge_index[0].astype(jnp.int32)
    col = edge_index[1].astype(jnp.int32)
    pad = E_PAD - E
    row2d = jnp.concatenate(
        [row, jnp.zeros((pad,), jnp.int32)]).reshape(NW * NB, B)
    col2d = jnp.concatenate(
        [col, jnp.full((pad,), DUMMY, jnp.int32)]).reshape(NW * NB, B)

    deg = _sc_degree(col2d)
    deg0 = deg[:N].reshape(N, 1)
    deg1 = deg[N_PAD:N_PAD + N].reshape(N, 1)

    dis, g1 = pl.pallas_call(
        _tc_head,
        out_shape=(jax.ShapeDtypeStruct((N, 1), jnp.float32),
                   jax.ShapeDtypeStruct((N, HID), jnp.float32)),
    )(deg0, deg1, x, W1)

    p1 = _sc_scatter16(g1, row2d, col2d)
    g2 = pl.pallas_call(
        _tc_mid,
        out_shape=jax.ShapeDtypeStruct((N, HID), jnp.float32),
    )(p1[:N], p1[N_PAD:N_PAD + N], g1, dis, b1.reshape(1, HID))

    p2 = _sc_scatter16(g2, row2d, col2d)
    out = pl.pallas_call(
        _tc_tail,
        out_shape=jax.ShapeDtypeStruct((N, NCLASS), jnp.float32),
    )(p2[:N], p2[N_PAD:N_PAD + N], g2, dis, W2, b2.reshape(1, NCLASS))
    return out


# speedup vs baseline: 32.4682x; 32.4682x over previous
import functools

import jax
import jax.numpy as jnp
from jax import lax
from jax.experimental import pallas as pl
from jax.experimental.pallas import tpu as pltpu
from jax.experimental.pallas import tpu_sc as plsc

N = 10000
E = 320000
NFEAT = 128
HID = 16
NCLASS = 40

NC = 2
NS = 16
NW = NC * NS
B = 128
NB = 80
PER_TILE = NB * B
E_PAD = PER_TILE * NW
NBUF = 16
assert E_PAD >= E and NB % NBUF == 0
N_PAD = 10240
RPT = N_PAD // NS
DUMMY = N

_mesh = plsc.VectorSubcoreMesh(
    core_axis_name="c", subcore_axis_name="s", num_cores=NC)
_sc_params = pltpu.CompilerParams(use_tc_tiling_on_sc=False)


@functools.partial(
    pl.kernel,
    out_type=jax.ShapeDtypeStruct((NC * N_PAD,), jnp.float32),
    mesh=_mesh,
    scratch_types=[
        pltpu.VMEM((NB, B), jnp.int32),
        pltpu.VMEM((B,), jnp.float32),
        pltpu.VMEM((RPT,), jnp.float32),
        pltpu.VMEM_SHARED((N_PAD,), jnp.float32),
        pltpu.SemaphoreType.DMA,
    ],
    compiler_params=_sc_params,
)
def _sc_degree(col_hbm, out_hbm, cidx, ones_v, bounce, acc, ssem):
    c = lax.axis_index("c")
    s = lax.axis_index("s")
    wid = c * NS + s

    one = jnp.ones((16,), jnp.float32)
    zero = jnp.zeros((16,), jnp.float32)
    for i in range(B // 16):
        ones_v[pl.ds(i * 16, 16)] = one

    def zb(i, carry):
        bounce[pl.ds(i * 16, 16)] = zero
        return carry
    lax.fori_loop(0, RPT // 16, zb, 0)
    pltpu.sync_copy(bounce, acc.at[pl.ds(s * RPT, RPT)])
    pltpu.sync_copy(col_hbm.at[pl.ds(wid * NB, NB)], cidx)
    plsc.subcore_barrier()

    def superblock(g, carry):
        j0 = g * NBUF
        hs = [pltpu.async_copy(ones_v, acc.at[cidx.at[j0 + b]], ssem, add=True)
              for b in range(NBUF)]
        for h in hs:
            h.wait()
        return carry
    lax.fori_loop(0, NB // NBUF, superblock, 0)

    plsc.subcore_barrier()
    pltpu.sync_copy(acc.at[pl.ds(s * RPT, RPT)], bounce)
    pltpu.sync_copy(bounce, out_hbm.at[pl.ds(c * N_PAD + s * RPT, RPT)])


@functools.partial(
    pl.kernel,
    out_type=jax.ShapeDtypeStruct((NC * N_PAD, HID), jnp.float32),
    mesh=_mesh,
    scratch_types=[
        pltpu.VMEM((NB, B), jnp.int32),
        pltpu.VMEM((NB, B), jnp.int32),
        pltpu.VMEM((NBUF * B, HID), jnp.float32),
        pltpu.VMEM((RPT, HID), jnp.float32),
        pltpu.VMEM_SHARED((N_PAD, HID), jnp.float32),
        pltpu.SemaphoreType.DMA,
        pltpu.SemaphoreType.DMA,
    ],
    compiler_params=_sc_params,
)
def _sc_scatter16(g_hbm, row_hbm, col_hbm, out_hbm,
                  ridx, cidx, rows, bounce, acc, gsem, ssem):
    c = lax.axis_index("c")
    s = lax.axis_index("s")
    wid = c * NS + s

    zero = jnp.zeros((16,), jnp.float32)

    def zb(i, carry):
        bounce[i] = zero
        return carry
    lax.fori_loop(0, RPT, zb, 0)
    pltpu.sync_copy(bounce, acc.at[pl.ds(s * RPT, RPT)])
    pltpu.sync_copy(row_hbm.at[pl.ds(wid * NB, NB)], ridx)
    pltpu.sync_copy(col_hbm.at[pl.ds(wid * NB, NB)], cidx)
    plsc.subcore_barrier()

    def superblock(g, carry):
        j0 = g * NBUF
        gathers = [pltpu.async_copy(g_hbm.at[ridx.at[j0 + b]],
                                    rows.at[pl.ds(b * B, B)], gsem)
                   for b in range(NBUF)]
        for h in gathers:
            h.wait()
        scatters = [pltpu.async_copy(rows.at[pl.ds(b * B, B)],
                                     acc.at[cidx.at[j0 + b]], ssem, add=True)
                    for b in range(NBUF)]
        for h in scatters:
            h.wait()
        return carry
    lax.fori_loop(0, NB // NBUF, superblock, 0)

    plsc.subcore_barrier()
    pltpu.sync_copy(acc.at[pl.ds(s * RPT, RPT)], bounce)
    pltpu.sync_copy(bounce, out_hbm.at[pl.ds(c * N_PAD + s * RPT, RPT)])


def _tc_head(deg0_ref, deg1_ref, x_ref, w1_ref, dis_ref, g1_ref):
    deg = deg0_ref[...] + deg1_ref[...] + 1.0
    dis = lax.rsqrt(deg)
    h1 = jnp.dot(x_ref[...], w1_ref[...], preferred_element_type=jnp.float32)
    dis_ref[...] = dis
    g1_ref[...] = h1 * dis


def _tc_mid(p0_ref, p1_ref, g1_ref, dis_ref, b1_ref, g2_ref):
    dis = dis_ref[...]
    s1 = dis * (p0_ref[...] + p1_ref[...] + g1_ref[...])
    a1 = jnp.maximum(s1 + b1_ref[...], 0.0)
    g2_ref[...] = a1 * dis


def _tc_tail(p0_ref, p1_ref, g2_ref, dis_ref, w2_ref, b2_ref, out_ref):
    s2 = dis_ref[...] * (p0_ref[...] + p1_ref[...] + g2_ref[...])
    logits = jnp.dot(s2, w2_ref[...],
                     preferred_element_type=jnp.float32) + b2_ref[...]
    m = jnp.max(logits, axis=1, keepdims=True)
    lse = jnp.log(jnp.sum(jnp.exp(logits - m), axis=1, keepdims=True)) + m
    out_ref[...] = logits - lse


def kernel(x, edge_index, W1, b1, W2, b2):
    row = edge_index[0].astype(jnp.int32)
    col = edge_index[1].astype(jnp.int32)
    pad = E_PAD - E
    row2d = jnp.concatenate(
        [row, jnp.zeros((pad,), jnp.int32)]).reshape(NW * NB, B)
    col2d = jnp.concatenate(
        [col, jnp.full((pad,), DUMMY, jnp.int32)]).reshape(NW * NB, B)

    deg = _sc_degree(col2d)
    deg0 = deg[:N].reshape(N, 1)
    deg1 = deg[N_PAD:N_PAD + N].reshape(N, 1)

    dis, g1 = pl.pallas_call(
        _tc_head,
        out_shape=(jax.ShapeDtypeStruct((N, 1), jnp.float32),
                   jax.ShapeDtypeStruct((N, HID), jnp.float32)),
    )(deg0, deg1, x, W1)

    p1 = _sc_scatter16(g1, row2d, col2d)
    g2 = pl.pallas_call(
        _tc_mid,
        out_shape=jax.ShapeDtypeStruct((N, HID), jnp.float32),
    )(p1[:N], p1[N_PAD:N_PAD + N], g1, dis, b1.reshape(1, HID))

    p2 = _sc_scatter16(g2, row2d, col2d)
    out = pl.pallas_call(
        _tc_tail,
        out_shape=jax.ShapeDtypeStruct((N, NCLASS), jnp.float32),
    )(p2[:N], p2[N_PAD:N_PAD + N], g2, dis, W2, b2.reshape(1, NCLASS))
    return out


# speedup vs baseline: 35.0583x; 1.0798x over previous
import functools

import jax
import jax.numpy as jnp
from jax import lax
from jax.experimental import pallas as pl
from jax.experimental.pallas import tpu as pltpu
from jax.experimental.pallas import tpu_sc as plsc

N = 10000
E = 320000
NFEAT = 128
HID = 16
NCLASS = 40

NC = 2
NS = 16
NW = NC * NS
B = 128
NB = 80
PER_TILE = NB * B
E_PAD = PER_TILE * NW
NBUF = 16
NSLOT = 16
AHEAD = 8
assert E_PAD >= E and NB % NBUF == 0
N_PAD = 10240
RPT = N_PAD // NS
DUMMY = N

_mesh = plsc.VectorSubcoreMesh(
    core_axis_name="c", subcore_axis_name="s", num_cores=NC)
_sc_params = pltpu.CompilerParams(use_tc_tiling_on_sc=False)


@functools.partial(
    pl.kernel,
    out_type=jax.ShapeDtypeStruct((NC * N_PAD,), jnp.float32),
    mesh=_mesh,
    scratch_types=[
        pltpu.VMEM((NB, B), jnp.int32),
        pltpu.VMEM((B,), jnp.float32),
        pltpu.VMEM((RPT,), jnp.float32),
        pltpu.VMEM_SHARED((N_PAD,), jnp.float32),
        pltpu.SemaphoreType.DMA,
    ],
    compiler_params=_sc_params,
)
def _sc_degree(col_hbm, out_hbm, cidx, ones_v, bounce, acc, ssem):
    c = lax.axis_index("c")
    s = lax.axis_index("s")
    wid = c * NS + s

    one = jnp.ones((16,), jnp.float32)
    zero = jnp.zeros((16,), jnp.float32)
    for i in range(B // 16):
        ones_v[pl.ds(i * 16, 16)] = one

    def zb(i, carry):
        bounce[pl.ds(i * 16, 16)] = zero
        return carry
    lax.fori_loop(0, RPT // 16, zb, 0)
    pltpu.sync_copy(bounce, acc.at[pl.ds(s * RPT, RPT)])
    pltpu.sync_copy(col_hbm.at[pl.ds(wid * NB, NB)], cidx)
    plsc.subcore_barrier()

    def superblock(g, carry):
        j0 = g * NBUF
        hs = [pltpu.async_copy(ones_v, acc.at[cidx.at[j0 + b]], ssem, add=True)
              for b in range(NBUF)]
        for h in hs:
            h.wait()
        return carry
    lax.fori_loop(0, NB // NBUF, superblock, 0)

    plsc.subcore_barrier()
    pltpu.sync_copy(acc.at[pl.ds(s * RPT, RPT)], bounce)
    pltpu.sync_copy(bounce, out_hbm.at[pl.ds(c * N_PAD + s * RPT, RPT)])


@functools.partial(
    pl.kernel,
    out_type=jax.ShapeDtypeStruct((NC * N_PAD, HID), jnp.float32),
    mesh=_mesh,
    scratch_types=[
        pltpu.VMEM((NB, B), jnp.int32),
        pltpu.VMEM((NB, B), jnp.int32),
        pltpu.VMEM((NSLOT * B, HID), jnp.float32),
        pltpu.VMEM((RPT, HID), jnp.float32),
        pltpu.VMEM_SHARED((N_PAD, HID), jnp.float32),
    ] + [pltpu.SemaphoreType.DMA] * NSLOT,
    compiler_params=_sc_params,
)
def _sc_scatter16(g_hbm, row_hbm, col_hbm, out_hbm,
                  ridx, cidx, rows, bounce, acc, *sems):
    c = lax.axis_index("c")
    s = lax.axis_index("s")
    wid = c * NS + s

    zero = jnp.zeros((16,), jnp.float32)

    def zb(i, carry):
        for k in range(8):
            bounce[i * 8 + k] = zero
        return carry
    lax.fori_loop(0, RPT // 8, zb, 0)
    pltpu.sync_copy(bounce, acc.at[pl.ds(s * RPT, RPT)])
    pltpu.sync_copy(row_hbm.at[pl.ds(wid * NB, NB)], ridx)
    pltpu.sync_copy(col_hbm.at[pl.ds(wid * NB, NB)], cidx)
    plsc.subcore_barrier()

    gh = [None] * NSLOT
    sh = [None] * NSLOT

    def fire_gather(j):
        sl = j % NSLOT
        if sh[sl] is not None:
            sh[sl].wait()
            sh[sl] = None
        gh[sl] = pltpu.async_copy(g_hbm.at[ridx.at[j]],
                                  rows.at[pl.ds(sl * B, B)], sems[sl])

    for j in range(min(AHEAD, NB)):
        fire_gather(j)
    for j in range(NB):
        sl = j % NSLOT
        if j + AHEAD < NB:
            fire_gather(j + AHEAD)
        gh[sl].wait()
        sh[sl] = pltpu.async_copy(rows.at[pl.ds(sl * B, B)],
                                  acc.at[cidx.at[j]], sems[sl], add=True)
    for h in sh:
        if h is not None:
            h.wait()

    plsc.subcore_barrier()
    pltpu.sync_copy(acc.at[pl.ds(s * RPT, RPT)], bounce)
    pltpu.sync_copy(bounce, out_hbm.at[pl.ds(c * N_PAD + s * RPT, RPT)])


def _tc_head(deg0_ref, deg1_ref, x_ref, w1_ref, dis_ref, g1_ref):
    deg = deg0_ref[...] + deg1_ref[...] + 1.0
    dis = lax.rsqrt(deg)
    h1 = jnp.dot(x_ref[...], w1_ref[...], preferred_element_type=jnp.float32)
    dis_ref[...] = dis
    g1_ref[...] = h1 * dis


def _tc_mid(p0_ref, p1_ref, g1_ref, dis_ref, b1_ref, g2_ref):
    dis = dis_ref[...]
    s1 = dis * (p0_ref[...] + p1_ref[...] + g1_ref[...])
    a1 = jnp.maximum(s1 + b1_ref[...], 0.0)
    g2_ref[...] = a1 * dis


def _tc_tail(p0_ref, p1_ref, g2_ref, dis_ref, w2_ref, b2_ref, out_ref):
    s2 = dis_ref[...] * (p0_ref[...] + p1_ref[...] + g2_ref[...])
    logits = jnp.dot(s2, w2_ref[...],
                     preferred_element_type=jnp.float32) + b2_ref[...]
    m = jnp.max(logits, axis=1, keepdims=True)
    lse = jnp.log(jnp.sum(jnp.exp(logits - m), axis=1, keepdims=True)) + m
    out_ref[...] = logits - lse


def kernel(x, edge_index, W1, b1, W2, b2):
    row = edge_index[0].astype(jnp.int32)
    col = edge_index[1].astype(jnp.int32)
    pad = E_PAD - E
    row2d = jnp.concatenate(
        [row, jnp.zeros((pad,), jnp.int32)]).reshape(NW * NB, B)
    col2d = jnp.concatenate(
        [col, jnp.full((pad,), DUMMY, jnp.int32)]).reshape(NW * NB, B)

    deg = _sc_degree(col2d)
    deg0 = deg[:N].reshape(N, 1)
    deg1 = deg[N_PAD:N_PAD + N].reshape(N, 1)

    dis, g1 = pl.pallas_call(
        _tc_head,
        out_shape=(jax.ShapeDtypeStruct((N, 1), jnp.float32),
                   jax.ShapeDtypeStruct((N, HID), jnp.float32)),
    )(deg0, deg1, x, W1)

    p1 = _sc_scatter16(g1, row2d, col2d)
    g2 = pl.pallas_call(
        _tc_mid,
        out_shape=jax.ShapeDtypeStruct((N, HID), jnp.float32),
    )(p1[:N], p1[N_PAD:N_PAD + N], g1, dis, b1.reshape(1, HID))

    p2 = _sc_scatter16(g2, row2d, col2d)
    out = pl.pallas_call(
        _tc_tail,
        out_shape=jax.ShapeDtypeStruct((N, NCLASS), jnp.float32),
    )(p2[:N], p2[N_PAD:N_PAD + N], g2, dis, W2, b2.reshape(1, NCLASS))
    return out


# speedup vs baseline: 57.3792x; 1.6367x over previous
import functools

import jax
import jax.numpy as jnp
from jax import lax
from jax.experimental import pallas as pl
from jax.experimental.pallas import tpu as pltpu
from jax.experimental.pallas import tpu_sc as plsc

N = 10000
E = 320000
NFEAT = 128
HID = 16
NCLASS = 40

NC = 2
NS = 16
NW = NC * NS
B = 128
NB = 80
PER_TILE = NB * B
E_PAD = PER_TILE * NW
NBUF = 16
NSLOT = 16
AHEAD = 8
assert E_PAD >= E and NB % NBUF == 0
N_PAD = 10240
RPT = N_PAD // NS
DUMMY = N

_mesh = plsc.VectorSubcoreMesh(
    core_axis_name="c", subcore_axis_name="s", num_cores=NC)
_sc_params = pltpu.CompilerParams(use_tc_tiling_on_sc=False)


@functools.partial(
    pl.kernel,
    out_type=jax.ShapeDtypeStruct((NC * N_PAD,), jnp.float32),
    mesh=_mesh,
    scratch_types=[
        pltpu.VMEM((NB, B), jnp.int32),
        pltpu.VMEM((B,), jnp.float32),
        pltpu.VMEM((RPT,), jnp.float32),
        pltpu.VMEM_SHARED((N_PAD,), jnp.float32),
        pltpu.SemaphoreType.DMA,
    ],
    compiler_params=_sc_params,
)
def _sc_degree(col_hbm, out_hbm, cidx, ones_v, bounce, acc, ssem):
    c = lax.axis_index("c")
    s = lax.axis_index("s")
    wid = c * NS + s

    one = jnp.ones((16,), jnp.float32)
    zero = jnp.zeros((16,), jnp.float32)
    for i in range(B // 16):
        ones_v[pl.ds(i * 16, 16)] = one

    def zb(i, carry):
        bounce[pl.ds(i * 16, 16)] = zero
        return carry
    lax.fori_loop(0, RPT // 16, zb, 0)
    pltpu.sync_copy(bounce, acc.at[pl.ds(s * RPT, RPT)])
    pltpu.sync_copy(col_hbm.at[pl.ds(wid * NB, NB)], cidx)
    plsc.subcore_barrier()

    def superblock(g, carry):
        j0 = g * NBUF
        hs = [pltpu.async_copy(ones_v, acc.at[cidx.at[j0 + b]], ssem, add=True)
              for b in range(NBUF)]
        for h in hs:
            h.wait()
        return carry
    lax.fori_loop(0, NB // NBUF, superblock, 0)

    plsc.subcore_barrier()
    pltpu.sync_copy(acc.at[pl.ds(s * RPT, RPT)], bounce)
    pltpu.sync_copy(bounce, out_hbm.at[pl.ds(c * N_PAD + s * RPT, RPT)])


@functools.partial(
    pl.kernel,
    out_type=jax.ShapeDtypeStruct((NC * N_PAD, HID), jnp.float32),
    mesh=_mesh,
    scratch_types=[
        pltpu.VMEM((NB, B), jnp.int32),
        pltpu.VMEM((NB, B), jnp.int32),
        pltpu.VMEM((NSLOT * B, HID), jnp.float32),
        pltpu.VMEM((RPT, HID), jnp.float32),
        pltpu.VMEM_SHARED((N_PAD, HID), jnp.float32),
    ] + [pltpu.SemaphoreType.DMA] * NSLOT,
    compiler_params=_sc_params,
)
def _sc_scatter16(g_hbm, row_hbm, col_hbm, out_hbm,
                  ridx, cidx, rows, bounce, acc, *sems):
    c = lax.axis_index("c")
    s = lax.axis_index("s")
    wid = c * NS + s

    zero = jnp.zeros((16,), jnp.float32)

    def zb(i, carry):
        for k in range(8):
            bounce[i * 8 + k] = zero
        return carry
    lax.fori_loop(0, RPT // 8, zb, 0)
    pltpu.sync_copy(bounce, acc.at[pl.ds(s * RPT, RPT)])
    pltpu.sync_copy(row_hbm.at[pl.ds(wid * NB, NB)], ridx)
    pltpu.sync_copy(col_hbm.at[pl.ds(wid * NB, NB)], cidx)
    plsc.subcore_barrier()

    gh = [None] * NSLOT
    sh = [None] * NSLOT

    def fire_gather(j):
        sl = j % NSLOT
        if sh[sl] is not None:
            sh[sl].wait()
            sh[sl] = None
        gh[sl] = pltpu.async_copy(g_hbm.at[ridx.at[j]],
                                  rows.at[pl.ds(sl * B, B)], sems[sl])

    for j in range(min(AHEAD, NB)):
        fire_gather(j)
    for j in range(NB):
        sl = j % NSLOT
        if j + AHEAD < NB:
            fire_gather(j + AHEAD)
        gh[sl].wait()
        sh[sl] = pltpu.async_copy(rows.at[pl.ds(sl * B, B)],
                                  acc.at[cidx.at[j]], sems[sl], add=True)
    for h in sh:
        if h is not None:
            h.wait()

    plsc.subcore_barrier()
    pltpu.sync_copy(acc.at[pl.ds(s * RPT, RPT)], bounce)
    pltpu.sync_copy(bounce, out_hbm.at[pl.ds(c * N_PAD + s * RPT, RPT)])


def _tc_head(deg0_ref, deg1_ref, x_ref, w1_ref, dis_ref, g1_ref):
    deg = deg0_ref[...] + deg1_ref[...] + 1.0
    dis = lax.rsqrt(deg)
    h1 = jnp.dot(x_ref[...], w1_ref[...], preferred_element_type=jnp.float32)
    dis_ref[...] = dis
    g1_ref[...] = h1 * dis


def _tc_mid(p0_ref, p1_ref, g1_ref, dis_ref, b1_ref, g2_ref):
    dis = dis_ref[...]
    s1 = dis * (p0_ref[...] + p1_ref[...] + g1_ref[...])
    a1 = jnp.maximum(s1 + b1_ref[...], 0.0)
    g2_ref[...] = a1 * dis


def _tc_tail(p0_ref, p1_ref, g2_ref, dis_ref, w2_ref, b2_ref, out_ref):
    s2 = dis_ref[...] * (p0_ref[...] + p1_ref[...] + g2_ref[...])
    logits = jnp.dot(s2, w2_ref[...],
                     preferred_element_type=jnp.float32) + b2_ref[...]
    m = jnp.max(logits, axis=1, keepdims=True)
    lse = jnp.log(jnp.sum(jnp.exp(logits - m), axis=1, keepdims=True)) + m
    out_ref[...] = logits - lse


def kernel(x, edge_index, W1, b1, W2, b2):
    row = edge_index[0].astype(jnp.int32)
    col = edge_index[1].astype(jnp.int32)
    pad = E_PAD - E
    pad_iota = jnp.arange(pad, dtype=jnp.int32)
    row2d = jnp.concatenate(
        [row, pad_iota % N]).reshape(NW * NB, B)
    col2d = jnp.concatenate(
        [col, DUMMY + pad_iota % (N_PAD - N)]).reshape(NW * NB, B)

    deg = _sc_degree(col2d)
    deg0 = deg[:N].reshape(N, 1)
    deg1 = deg[N_PAD:N_PAD + N].reshape(N, 1)

    dis, g1 = pl.pallas_call(
        _tc_head,
        out_shape=(jax.ShapeDtypeStruct((N, 1), jnp.float32),
                   jax.ShapeDtypeStruct((N, HID), jnp.float32)),
    )(deg0, deg1, x, W1)

    p1 = _sc_scatter16(g1, row2d, col2d)
    g2 = pl.pallas_call(
        _tc_mid,
        out_shape=jax.ShapeDtypeStruct((N, HID), jnp.float32),
    )(p1[:N], p1[N_PAD:N_PAD + N], g1, dis, b1.reshape(1, HID))

    p2 = _sc_scatter16(g2, row2d, col2d)
    out = pl.pallas_call(
        _tc_tail,
        out_shape=jax.ShapeDtypeStruct((N, NCLASS), jnp.float32),
    )(p2[:N], p2[N_PAD:N_PAD + N], g2, dis, W2, b2.reshape(1, NCLASS))
    return out
